# baseline (speedup 1.0000x reference)
import functools

import jax
import jax.numpy as jnp
from jax import lax
from jax.experimental import pallas as pl
from jax.experimental.pallas import tpu as pltpu
from jax.experimental.pallas import tpu_sc as plsc


def _bias_body(ts_ref, ch_ref, pw_ref, cw_ref, o_ref):
    n_tok = jnp.sum(ts_ref[...])
    n_ch = jnp.sum(ch_ref[...])
    max_n, emb = pw_ref.shape
    max_c = cw_ref.shape[0]
    rows_p = lax.broadcasted_iota(jnp.int32, (max_n, emb), 0)
    sel_p = jnp.where(rows_p < n_tok, pw_ref[...], pw_ref[max_n - 1:max_n, :])
    rows_c = lax.broadcasted_iota(jnp.int32, (max_c, emb), 0)
    sel_c = jnp.where(rows_c < n_ch, cw_ref[...], cw_ref[max_c - 1:max_c, :])
    o_ref[...] = sel_c[:, None, :] + sel_p[None, :, :]


_NC, _NS, _L = 2, 16, 16


def _make_sc_body(bs, max_c, max_n, emb, csplit):
    nw = _NC * _NS
    bpw = bs // nw
    cw = max_c // csplit
    nch = bpw * csplit
    nvec = emb // _L

    def _body(bias_hbm, x_hbm, o_hbm, bias_v, xin, xout, isems, osems):
        wid = lax.axis_index("s") * _NC + lax.axis_index("c")
        base = wid * bpw
        pltpu.sync_copy(bias_hbm, bias_v)

        def in_cp(t, k):
            b = base + t // csplit
            c0 = (t % csplit) * cw
            return pltpu.make_async_copy(
                x_hbm.at[b, pl.ds(c0, cw)], xin.at[k], isems.at[k])

        def out_cp(t, k):
            b = base + t // csplit
            c0 = (t % csplit) * cw
            return pltpu.make_async_copy(
                xout.at[k], o_hbm.at[b, pl.ds(c0, cw)], osems.at[k])

        in_cp(0, 0).start()

        def step(t, carry):
            k = t % 2
            in_cp(t, k).wait()

            @pl.when(t + 1 < nch)
            def _():
                in_cp(t + 1, 1 - k).start()

            @pl.when(t >= 2)
            def _():
                out_cp(t - 2, k).wait()

            c0 = (t % csplit) * cw

            for ci in range(cw):
                for n in range(max_n):
                    for j in range(nvec):
                        sl = pl.ds(j * _L, _L)
                        xout[k, ci, n, sl] = (
                            xin[k, ci, n, sl] + bias_v[c0 + ci, n, sl])
            out_cp(t, k).start()
            return carry

        lax.fori_loop(0, nch, step, 0)
        out_cp(nch - 2, (nch - 2) % 2).wait()
        out_cp(nch - 1, (nch - 1) % 2).wait()

    return _body


@functools.partial(jax.jit, static_argnames=("csplit",))
def _run(x, ts_i, ch_i, patch_pos_w, ch_pos_w, csplit=3):
    bs, max_c, max_n, emb = x.shape
    bias = pl.pallas_call(
        _bias_body,
        out_shape=jax.ShapeDtypeStruct((max_c, max_n, emb), x.dtype),
    )(ts_i, ch_i, patch_pos_w, ch_pos_w)
    cw = max_c // csplit
    sc_add = functools.partial(
        pl.kernel,
        out_type=jax.ShapeDtypeStruct((bs, max_c, max_n, emb), x.dtype),
        mesh=plsc.VectorSubcoreMesh(core_axis_name="c", subcore_axis_name="s"),
        scratch_types=[
            pltpu.VMEM((max_c, max_n, emb), x.dtype),
            pltpu.VMEM((2, cw, max_n, emb), x.dtype),
            pltpu.VMEM((2, cw, max_n, emb), x.dtype),
            pltpu.SemaphoreType.DMA((2,)),
            pltpu.SemaphoreType.DMA((2,)),
        ],
    )(_make_sc_body(bs, max_c, max_n, emb, csplit))
    return sc_add(bias, x)


def kernel(x, ts_token_mask, ch_mask, patch_pos_w, ch_pos_w):
    ts_i = ts_token_mask.astype(jnp.int32)
    ch_i = ch_mask.astype(jnp.int32)
    return _run(x, ts_i, ch_i, patch_pos_w, ch_pos_w)


# speedup vs baseline: 1.8748x; 1.8748x over previous
import functools

import jax
import jax.numpy as jnp
from jax import lax
from jax.experimental import pallas as pl
from jax.experimental.pallas import tpu as pltpu
from jax.experimental.pallas import tpu_sc as plsc

_NC, _NS, _L = 2, 16, 16


def _make_bias_body(max_c, max_n, emb):
    nvec = emb // _L

    def _body(ts_ref, ch_ref, pw_hbm, cw_hbm, o_hbm, tsv, chv, pwv, cwv,
              ov):
        wid = lax.axis_index("s") * _NC + lax.axis_index("c")

        @pl.when(wid < max_c)
        def _():
            pltpu.sync_copy(ts_ref, tsv)
            pltpu.sync_copy(ch_ref, chv)
            pltpu.sync_copy(pw_hbm, pwv)
            pltpu.sync_copy(cw_hbm, cwv)
            tsvec = tsv[...]
            chvec0 = chv[pl.ds(0, _L)]
            chvec1 = chv[pl.ds(_L, _L)]
            n_tok = sum(tsvec[i] for i in range(_L))
            n_ch = sum(chvec0[i] + chvec1[i] for i in range(_L))
            cidx = jnp.where(wid < n_ch, wid, max_c - 1)
            for j in range(nvec):
                sl = pl.ds(j * _L, _L)
                cvec = cwv[cidx, sl]
                for n in range(max_n):
                    pidx = jnp.where(n < n_tok, n, max_n - 1)
                    ov[n, sl] = cvec + pwv[pidx, sl]
            pltpu.sync_copy(ov, o_hbm.at[wid])

    return _body


def _make_stream_body(bs, bb, ring):
    nb = bs // bb

    def _body(b_ref, x_hbm, o_hbm, xbuf, obuf, in_sems, out_sems):
        def in_copy(i, k):
            return pltpu.make_async_copy(
                x_hbm.at[pl.ds(i * bb, bb)], xbuf.at[k], in_sems.at[k])

        def out_copy(i, k):
            return pltpu.make_async_copy(
                obuf.at[k], o_hbm.at[pl.ds(i * bb, bb)], out_sems.at[k])

        bias = b_ref[...][None]
        for i in range(min(ring, nb)):
            in_copy(i, i % ring).start()
        for i in range(nb):
            k = i % ring
            in_copy(i, k).wait()
            if i >= ring:
                out_copy(i - ring, k).wait()
            obuf[k] = xbuf[k] + bias
            out_copy(i, k).start()
            if i + ring < nb:
                in_copy(i + ring, k).start()
        for i in range(max(nb - ring, 0), nb):
            out_copy(i, i % ring).wait()

    return _body


@functools.partial(jax.jit, static_argnames=("bb", "ring"))
def _run(x, ts_i, ch_i, patch_pos_w, ch_pos_w, bb=16, ring=8):
    bs, max_c, max_n, emb = x.shape
    bias_fn = pl.kernel(
        _make_bias_body(max_c, max_n, emb),
        out_type=jax.ShapeDtypeStruct((max_c, max_n, emb), x.dtype),
        mesh=plsc.VectorSubcoreMesh(core_axis_name="c", subcore_axis_name="s"),
        scratch_types=[
            pltpu.VMEM((_L,), jnp.int32),
            pltpu.VMEM((2 * _L,), jnp.int32),
            pltpu.VMEM((max_n, emb), x.dtype),
            pltpu.VMEM((max_c, emb), x.dtype),
            pltpu.VMEM((max_n, emb), x.dtype),
        ],
    )
    bias = bias_fn(ts_i, ch_i, patch_pos_w, ch_pos_w)
    out = pl.pallas_call(
        _make_stream_body(bs, bb, ring),
        in_specs=[
            pl.BlockSpec(memory_space=pltpu.VMEM),
            pl.BlockSpec(memory_space=pl.ANY),
        ],
        out_specs=pl.BlockSpec(memory_space=pl.ANY),
        out_shape=jax.ShapeDtypeStruct((bs, max_c, max_n, emb), x.dtype),
        scratch_shapes=[
            pltpu.VMEM((ring, bb, max_c, max_n, emb), x.dtype),
            pltpu.VMEM((ring, bb, max_c, max_n, emb), x.dtype),
            pltpu.SemaphoreType.DMA((ring,)),
            pltpu.SemaphoreType.DMA((ring,)),
        ],
    )(bias, x)
    return out


def kernel(x, ts_token_mask, ch_mask, patch_pos_w, ch_pos_w):
    ts_i = jnp.pad(ts_token_mask.astype(jnp.int32),
                   ((0, 0), (0, _L - ts_token_mask.shape[1]))).reshape(_L)
    ch_i = jnp.pad(ch_mask.astype(jnp.int32),
                   ((0, 0), (0, 2 * _L - ch_mask.shape[1]))).reshape(2 * _L)
    return _run(x, ts_i, ch_i, patch_pos_w, ch_pos_w)
